# baseline (speedup 1.0000x reference)
import functools

import jax
import jax.numpy as jnp
from jax import lax
from jax.experimental import pallas as pl
from jax.experimental.pallas import tpu as pltpu
from jax.experimental.pallas import tpu_sc as plsc

V1 = 10000
K1 = 32
V2 = 2500
K2 = 32
F = 128
KI = 4
E1 = V1 * K1
NI = V1 * KI

_WPH = 25
_CN = 40
_SEGS = (2000, 3000, 5000)

_SELU_ALPHA = 1.6732632423543772
_SELU_SCALE = 1.0507009873554805


def _edge_to_node_body(euvt_ref, ea2_ref, v_ref):
    ea = ea2_ref[...]
    a0 = euvt_ref[:, :, 0:1]
    a1 = euvt_ref[:, :, 1:2]
    r0 = jnp.sum(ea * a0, axis=1, keepdims=True)
    r1 = jnp.sum(ea * a1, axis=1, keepdims=True)
    v_ref[...] = jnp.concatenate([r0, r1], axis=1)


def _edge_to_node(euvt, ea2_3d):
    B2 = 125
    grid = V2 // B2
    return pl.pallas_call(
        _edge_to_node_body,
        grid=(grid,),
        in_specs=[
            pl.BlockSpec((B2, K2, 2), lambda i: (i, 0, 0)),
            pl.BlockSpec((B2, K2, F), lambda i: (i, 0, 0)),
        ],
        out_specs=pl.BlockSpec((B2, 2, F), lambda i: (i, 0, 0)),
        out_shape=jax.ShapeDtypeStruct((V2, 2, F), jnp.float32),
    )(euvt, ea2_3d)


def _knn_interp(v2d, idx_h, w_h, seg):
    mesh = plsc.VectorSubcoreMesh(core_axis_name="c", subcore_axis_name="s")
    npw = seg // _WPH
    nch = npw // _CN
    rpw = npw * KI
    rch = _CN * KI

    @functools.partial(
        pl.kernel,
        mesh=mesh,
        out_type=jax.ShapeDtypeStruct((seg, 2 * F), jnp.float32),
        scratch_types=[
            pltpu.VMEM((rpw,), jnp.int32),
            pltpu.VMEM((rpw,), jnp.float32),
            pltpu.VMEM((2, rch, 2 * F), jnp.float32),
            pltpu.VMEM((2, _CN, 2 * F), jnp.float32),
            pltpu.SemaphoreType.DMA((2,)),
            pltpu.SemaphoreType.DMA((2,)),
        ],
    )
    def k(v_hbm, idx_hbm, w_hbm, out_hbm, idx_v, w_v, rows_v, out_v, semg, sems):
        wid = lax.axis_index("s") * 2 + lax.axis_index("c")

        @pl.when(wid < _WPH)
        def _():
            pltpu.sync_copy(idx_hbm.at[pl.ds(wid * rpw, rpw)], idx_v)
            pltpu.sync_copy(w_hbm.at[pl.ds(wid * rpw, rpw)], w_v)

            def gather(t, b):
                return pltpu.async_copy(
                    v_hbm.at[idx_v.at[pl.ds(t * rch, rch)]],
                    rows_v.at[b], semg.at[b])

            gathers = [gather(0, 0), None]
            stores = [None, None]
            for t in range(nch):
                cb = t % 2
                nb = (t + 1) % 2
                if t + 1 < nch:
                    gathers[nb] = gather(t + 1, nb)
                gathers[cb].wait()
                if stores[cb] is not None:
                    stores[cb].wait()
                    stores[cb] = None

                def body(g, _):
                    wvec = w_v[pl.ds(t * rch + 16 * g, 16)]
                    for j in range(4):
                        i = 4 * g + j
                        w0, w1, w2, w3 = (wvec[4 * j + m] for m in range(4))
                        inv = jnp.ones((16,), jnp.float32) / jnp.broadcast_to(
                            w0 + w1 + w2 + w3, (16,))
                        for c in range(2 * F // 16):
                            s = pl.ds(c * 16, 16)
                            acc = (w0 * rows_v[cb, 4 * i, s]
                                   + w1 * rows_v[cb, 4 * i + 1, s]
                                   + w2 * rows_v[cb, 4 * i + 2, s]
                                   + w3 * rows_v[cb, 4 * i + 3, s])
                            out_v[cb, i, s] = acc * inv
                    return 0

                lax.fori_loop(0, _CN // 4, body, 0)
                stores[cb] = pltpu.async_copy(
                    out_v.at[cb], out_hbm.at[pl.ds(wid * npw + t * _CN, _CN)],
                    sems.at[cb])
            for st in stores:
                if st is not None:
                    st.wait()

    return k(v2d, idx_h, w_h)


def _selu(x):
    return _SELU_SCALE * jnp.where(x > 0, x, _SELU_ALPHA * (jnp.exp(x) - 1.0))


def _mlp_body(*refs, nb):
    if len(refs) == 14:
        refs = refs[1:]
    (v1_ref, euv_ref, ea1_ref, w1a_ref, w1b_ref, b1_ref, w2_ref,
     b2_ref, w3_ref, b3_ref, g_ref, bt_ref, out_ref) = refs
    ne = nb * K1
    v1 = v1_ref[...]
    va = jnp.broadcast_to(v1[:, :F].reshape(nb, 1, F), (nb, K1, F)).reshape(ne, F)
    vb = jnp.broadcast_to(v1[:, F:].reshape(nb, 1, F), (nb, K1, F)).reshape(ne, F)
    euv = euv_ref[...]
    e1 = euv[:, 0:1] * va + euv[:, 1:2] * vb
    x1 = ea1_ref[...]
    h = jnp.dot(x1, w1a_ref[...], preferred_element_type=jnp.float32)
    h += jnp.dot(e1, w1b_ref[...], preferred_element_type=jnp.float32)
    h = _selu(h + b1_ref[...])
    h = _selu(jnp.dot(h, w2_ref[...], preferred_element_type=jnp.float32) + b2_ref[...])
    h = jnp.dot(h, w3_ref[...], preferred_element_type=jnp.float32) + b3_ref[...]
    mu = jnp.mean(h, axis=1, keepdims=True)
    d = h - mu
    var = jnp.mean(d * d, axis=1, keepdims=True)
    out_ref[...] = x1 + d * jax.lax.rsqrt(var + 1e-5) * g_ref[...] + bt_ref[...]


def _mlp_seg(v1_s, euv1, ea1, consts, grid, off, prev_out=None):
    nb = 200
    ne = nb * K1
    const = lambda i: (0, 0)
    in_specs = [
        pl.BlockSpec((nb, 2 * F), lambda i: (i, 0)),
        pl.BlockSpec((ne, 2), lambda i: (i + off, 0)),
        pl.BlockSpec((ne, F), lambda i: (i + off, 0)),
        pl.BlockSpec((F, F), const),
        pl.BlockSpec((F, F), const),
        pl.BlockSpec((1, F), const),
        pl.BlockSpec((F, F), const),
        pl.BlockSpec((1, F), const),
        pl.BlockSpec((F, F), const),
        pl.BlockSpec((1, F), const),
        pl.BlockSpec((1, F), const),
        pl.BlockSpec((1, F), const),
    ]
    args = [v1_s, euv1, ea1, *consts]
    kwargs = {}
    if prev_out is not None:
        in_specs = [pl.BlockSpec(memory_space=pl.ANY)] + in_specs
        args = [prev_out] + args
        kwargs["input_output_aliases"] = {0: 0}
    return pl.pallas_call(
        functools.partial(_mlp_body, nb=nb),
        grid=(grid,),
        in_specs=in_specs,
        out_specs=pl.BlockSpec((ne, F), lambda i: (i + off, 0)),
        out_shape=jax.ShapeDtypeStruct((E1, F), jnp.float32),
        **kwargs,
    )(*args)


def kernel(pos, y_idx_21, x_idx_21, weights_21, edge_attr2, edge_index2,
           edgeUnitVectorInverse2, coarse_mask2, edge_attr1, edge_index1,
           edgeUnitVector1, W1, b1, W2, b2, W3, b3, gamma, beta):
    ea2_3d = edge_attr2.reshape(V2, K2, F)
    euvt = edgeUnitVectorInverse2.transpose(0, 2, 1)
    v = _edge_to_node(euvt, ea2_3d)
    v2d = v.reshape(V2, 2 * F)

    idx = x_idx_21.astype(jnp.int32)
    w = weights_21.reshape(-1)
    v1s = []
    n0 = 0
    for seg in _SEGS:
        v1s.append(_knn_interp(v2d, idx[n0 * KI:(n0 + seg) * KI],
                               w[n0 * KI:(n0 + seg) * KI], seg))
        n0 += seg

    consts = (W1[:F], W1[F:], b1.reshape(1, F), W2, b2.reshape(1, F),
              W3, b3.reshape(1, F), gamma.reshape(1, F), beta.reshape(1, F))
    out = None
    off = 0
    for v1_s, seg in zip(v1s, _SEGS):
        grid = seg // 200
        out = _mlp_seg(v1_s, edgeUnitVector1, edge_attr1, consts, grid, off,
                       prev_out=out)
        off += grid
    return out


# speedup vs baseline: 1.0072x; 1.0072x over previous
import functools

import jax
import jax.numpy as jnp
from jax import lax
from jax.experimental import pallas as pl
from jax.experimental.pallas import tpu as pltpu
from jax.experimental.pallas import tpu_sc as plsc

V1 = 10000
K1 = 32
V2 = 2500
K2 = 32
F = 128
KI = 4
E1 = V1 * K1
NI = V1 * KI

_WPH = 25
_CN = 40
_SEGS = (3000, 7000)

_SELU_ALPHA = 1.6732632423543772
_SELU_SCALE = 1.0507009873554805


def _edge_to_node_body(euvt_ref, ea2_ref, v_ref):
    ea = ea2_ref[...]
    a0 = euvt_ref[:, :, 0:1]
    a1 = euvt_ref[:, :, 1:2]
    r0 = jnp.sum(ea * a0, axis=1, keepdims=True)
    r1 = jnp.sum(ea * a1, axis=1, keepdims=True)
    v_ref[...] = jnp.concatenate([r0, r1], axis=1)


def _edge_to_node(euvt, ea2_3d):
    B2 = 125
    grid = V2 // B2
    return pl.pallas_call(
        _edge_to_node_body,
        grid=(grid,),
        in_specs=[
            pl.BlockSpec((B2, K2, 2), lambda i: (i, 0, 0)),
            pl.BlockSpec((B2, K2, F), lambda i: (i, 0, 0)),
        ],
        out_specs=pl.BlockSpec((B2, 2, F), lambda i: (i, 0, 0)),
        out_shape=jax.ShapeDtypeStruct((V2, 2, F), jnp.float32),
    )(euvt, ea2_3d)


def _knn_interp(v2d, idx_h, w_h, seg):
    mesh = plsc.VectorSubcoreMesh(core_axis_name="c", subcore_axis_name="s")
    npw = seg // _WPH
    nch = npw // _CN
    rpw = npw * KI
    rch = _CN * KI

    @functools.partial(
        pl.kernel,
        mesh=mesh,
        out_type=jax.ShapeDtypeStruct((seg, 2 * F), jnp.float32),
        scratch_types=[
            pltpu.VMEM((rpw,), jnp.int32),
            pltpu.VMEM((rpw,), jnp.float32),
            pltpu.VMEM((2, rch, 2 * F), jnp.float32),
            pltpu.VMEM((2, _CN, 2 * F), jnp.float32),
            pltpu.SemaphoreType.DMA((2,)),
            pltpu.SemaphoreType.DMA((2,)),
        ],
    )
    def k(v_hbm, idx_hbm, w_hbm, out_hbm, idx_v, w_v, rows_v, out_v, semg, sems):
        wid = lax.axis_index("s") * 2 + lax.axis_index("c")

        @pl.when(wid < _WPH)
        def _():
            pltpu.sync_copy(idx_hbm.at[pl.ds(wid * rpw, rpw)], idx_v)
            pltpu.sync_copy(w_hbm.at[pl.ds(wid * rpw, rpw)], w_v)

            def gather(t, b):
                return pltpu.async_copy(
                    v_hbm.at[idx_v.at[pl.ds(t * rch, rch)]],
                    rows_v.at[b], semg.at[b])

            gathers = [gather(0, 0), None]
            stores = [None, None]
            for t in range(nch):
                cb = t % 2
                nb = (t + 1) % 2
                if t + 1 < nch:
                    gathers[nb] = gather(t + 1, nb)
                gathers[cb].wait()
                if stores[cb] is not None:
                    stores[cb].wait()
                    stores[cb] = None

                def body(g, _):
                    wvec = w_v[pl.ds(t * rch + 16 * g, 16)]
                    for j in range(4):
                        i = 4 * g + j
                        w0, w1, w2, w3 = (wvec[4 * j + m] for m in range(4))
                        inv = jnp.ones((16,), jnp.float32) / jnp.broadcast_to(
                            w0 + w1 + w2 + w3, (16,))
                        for c in range(2 * F // 16):
                            s = pl.ds(c * 16, 16)
                            acc = (w0 * rows_v[cb, 4 * i, s]
                                   + w1 * rows_v[cb, 4 * i + 1, s]
                                   + w2 * rows_v[cb, 4 * i + 2, s]
                                   + w3 * rows_v[cb, 4 * i + 3, s])
                            out_v[cb, i, s] = acc * inv
                    return 0

                lax.fori_loop(0, _CN // 4, body, 0)
                stores[cb] = pltpu.async_copy(
                    out_v.at[cb], out_hbm.at[pl.ds(wid * npw + t * _CN, _CN)],
                    sems.at[cb])
            for st in stores:
                if st is not None:
                    st.wait()

    return k(v2d, idx_h, w_h)


def _selu(x):
    return _SELU_SCALE * jnp.where(x > 0, x, _SELU_ALPHA * (jnp.exp(x) - 1.0))


def _mlp_body(*refs, nb):
    if len(refs) == 14:
        refs = refs[1:]
    (v1_ref, euv_ref, ea1_ref, w1a_ref, w1b_ref, b1_ref, w2_ref,
     b2_ref, w3_ref, b3_ref, g_ref, bt_ref, out_ref) = refs
    ne = nb * K1
    v1 = v1_ref[...]
    va = jnp.broadcast_to(v1[:, :F].reshape(nb, 1, F), (nb, K1, F)).reshape(ne, F)
    vb = jnp.broadcast_to(v1[:, F:].reshape(nb, 1, F), (nb, K1, F)).reshape(ne, F)
    euv = euv_ref[...]
    e1 = euv[:, 0:1] * va + euv[:, 1:2] * vb
    x1 = ea1_ref[...]
    h = jnp.dot(x1, w1a_ref[...], preferred_element_type=jnp.float32)
    h += jnp.dot(e1, w1b_ref[...], preferred_element_type=jnp.float32)
    h = _selu(h + b1_ref[...])
    h = _selu(jnp.dot(h, w2_ref[...], preferred_element_type=jnp.float32) + b2_ref[...])
    h = jnp.dot(h, w3_ref[...], preferred_element_type=jnp.float32) + b3_ref[...]
    mu = jnp.mean(h, axis=1, keepdims=True)
    d = h - mu
    var = jnp.mean(d * d, axis=1, keepdims=True)
    out_ref[...] = x1 + d * jax.lax.rsqrt(var + 1e-5) * g_ref[...] + bt_ref[...]


def _mlp_seg(v1_s, euv1, ea1, consts, grid, off, prev_out=None):
    nb = 200
    ne = nb * K1
    const = lambda i: (0, 0)
    in_specs = [
        pl.BlockSpec((nb, 2 * F), lambda i: (i, 0)),
        pl.BlockSpec((ne, 2), lambda i: (i + off, 0)),
        pl.BlockSpec((ne, F), lambda i: (i + off, 0)),
        pl.BlockSpec((F, F), const),
        pl.BlockSpec((F, F), const),
        pl.BlockSpec((1, F), const),
        pl.BlockSpec((F, F), const),
        pl.BlockSpec((1, F), const),
        pl.BlockSpec((F, F), const),
        pl.BlockSpec((1, F), const),
        pl.BlockSpec((1, F), const),
        pl.BlockSpec((1, F), const),
    ]
    args = [v1_s, euv1, ea1, *consts]
    kwargs = {}
    if prev_out is not None:
        in_specs = [pl.BlockSpec(memory_space=pl.ANY)] + in_specs
        args = [prev_out] + args
        kwargs["input_output_aliases"] = {0: 0}
    return pl.pallas_call(
        functools.partial(_mlp_body, nb=nb),
        grid=(grid,),
        in_specs=in_specs,
        out_specs=pl.BlockSpec((ne, F), lambda i: (i + off, 0)),
        out_shape=jax.ShapeDtypeStruct((E1, F), jnp.float32),
        **kwargs,
    )(*args)


def kernel(pos, y_idx_21, x_idx_21, weights_21, edge_attr2, edge_index2,
           edgeUnitVectorInverse2, coarse_mask2, edge_attr1, edge_index1,
           edgeUnitVector1, W1, b1, W2, b2, W3, b3, gamma, beta):
    ea2_3d = edge_attr2.reshape(V2, K2, F)
    euvt = edgeUnitVectorInverse2.transpose(0, 2, 1)
    v = _edge_to_node(euvt, ea2_3d)
    v2d = v.reshape(V2, 2 * F)

    idx = x_idx_21.astype(jnp.int32)
    w = weights_21.reshape(-1)
    v1s = []
    n0 = 0
    for seg in _SEGS:
        v1s.append(_knn_interp(v2d, idx[n0 * KI:(n0 + seg) * KI],
                               w[n0 * KI:(n0 + seg) * KI], seg))
        n0 += seg

    consts = (W1[:F], W1[F:], b1.reshape(1, F), W2, b2.reshape(1, F),
              W3, b3.reshape(1, F), gamma.reshape(1, F), beta.reshape(1, F))
    out = None
    off = 0
    for v1_s, seg in zip(v1s, _SEGS):
        grid = seg // 200
        out = _mlp_seg(v1_s, edgeUnitVector1, edge_attr1, consts, grid, off,
                       prev_out=out)
        off += grid
    return out


# speedup vs baseline: 1.0546x; 1.0471x over previous
import functools

import jax
import jax.numpy as jnp
from jax import lax
from jax.experimental import pallas as pl
from jax.experimental.pallas import tpu as pltpu
from jax.experimental.pallas import tpu_sc as plsc

V1 = 10000
K1 = 32
V2 = 2500
K2 = 32
F = 128
KI = 4
E1 = V1 * K1
NI = V1 * KI

_WPH = 25
_CN = 40
_SEGS = (3000, 7000)

_SELU_ALPHA = 1.6732632423543772
_SELU_SCALE = 1.0507009873554805


def _edge_to_node_body(euvt_ref, ea2_ref, v_ref):
    ea = ea2_ref[...]
    a0 = euvt_ref[:, :, 0:1]
    a1 = euvt_ref[:, :, 1:2]
    r0 = jnp.sum(ea * a0, axis=1, keepdims=True)
    r1 = jnp.sum(ea * a1, axis=1, keepdims=True)
    v_ref[...] = jnp.concatenate([r0, r1], axis=1)


def _edge_to_node(euvt, ea2_3d):
    B2 = 125
    grid = V2 // B2
    return pl.pallas_call(
        _edge_to_node_body,
        grid=(grid,),
        in_specs=[
            pl.BlockSpec((B2, K2, 2), lambda i: (i, 0, 0)),
            pl.BlockSpec((B2, K2, F), lambda i: (i, 0, 0)),
        ],
        out_specs=pl.BlockSpec((B2, 2, F), lambda i: (i, 0, 0)),
        out_shape=jax.ShapeDtypeStruct((V2, 2, F), jnp.float32),
    )(euvt, ea2_3d)


def _knn_interp(v2d, idx_h, w_h, seg):
    mesh = plsc.VectorSubcoreMesh(core_axis_name="c", subcore_axis_name="s")
    npw = seg // _WPH
    nch = npw // _CN
    rpw = npw * KI
    rch = _CN * KI

    @functools.partial(
        pl.kernel,
        mesh=mesh,
        out_type=jax.ShapeDtypeStruct((seg, 2 * F), jnp.float32),
        scratch_types=[
            pltpu.VMEM((rpw,), jnp.int32),
            pltpu.VMEM((rpw,), jnp.float32),
            pltpu.VMEM((2, rch, 2 * F), jnp.float32),
            pltpu.VMEM((2, _CN, 2 * F), jnp.float32),
            pltpu.SemaphoreType.DMA((2,)),
            pltpu.SemaphoreType.DMA((2,)),
        ],
    )
    def k(v_hbm, idx_hbm, w_hbm, out_hbm, idx_v, w_v, rows_v, out_v, semg, sems):
        wid = lax.axis_index("s") * 2 + lax.axis_index("c")

        @pl.when(wid < _WPH)
        def _():
            pltpu.sync_copy(idx_hbm.at[pl.ds(wid * rpw, rpw)], idx_v)
            pltpu.sync_copy(w_hbm.at[pl.ds(wid * rpw, rpw)], w_v)

            def gather(t, b):
                return pltpu.async_copy(
                    v_hbm.at[idx_v.at[pl.ds(t * rch, rch)]],
                    rows_v.at[b], semg.at[b])

            gathers = [gather(0, 0), None]
            stores = [None, None]
            for t in range(nch):
                cb = t % 2
                nb = (t + 1) % 2
                if t + 1 < nch:
                    gathers[nb] = gather(t + 1, nb)
                gathers[cb].wait()
                if stores[cb] is not None:
                    stores[cb].wait()
                    stores[cb] = None

                def body(g, _):
                    wvec = w_v[pl.ds(t * rch + 16 * g, 16)]
                    for j in range(4):
                        i = 4 * g + j
                        w0, w1, w2, w3 = (wvec[4 * j + m] for m in range(4))
                        inv = jnp.ones((16,), jnp.float32) / jnp.broadcast_to(
                            w0 + w1 + w2 + w3, (16,))
                        for c in range(2 * F // 16):
                            s = pl.ds(c * 16, 16)
                            acc = (w0 * rows_v[cb, 4 * i, s]
                                   + w1 * rows_v[cb, 4 * i + 1, s]
                                   + w2 * rows_v[cb, 4 * i + 2, s]
                                   + w3 * rows_v[cb, 4 * i + 3, s])
                            out_v[cb, i, s] = acc * inv
                    return 0

                lax.fori_loop(0, _CN // 4, body, 0)
                stores[cb] = pltpu.async_copy(
                    out_v.at[cb], out_hbm.at[pl.ds(wid * npw + t * _CN, _CN)],
                    sems.at[cb])
            for st in stores:
                if st is not None:
                    st.wait()

    return k(v2d, idx_h, w_h)


def _selu(x):
    return _SELU_SCALE * jnp.where(x > 0, x, _SELU_ALPHA * (jnp.exp(x) - 1.0))


def _mlp_body(*refs, nb):
    if len(refs) == 14:
        refs = refs[1:]
    (v1_ref, euv_ref, ea1_ref, w1a_ref, w1b_ref, b1_ref, w2_ref,
     b2_ref, w3_ref, b3_ref, g_ref, bt_ref, out_ref) = refs
    ne = nb * K1
    v1 = v1_ref[...]
    p = jnp.dot(v1[:, :F], w1b_ref[...], preferred_element_type=jnp.float32)
    q = jnp.dot(v1[:, F:], w1b_ref[...], preferred_element_type=jnp.float32)
    pr = jnp.broadcast_to(p.reshape(nb, 1, F), (nb, K1, F)).reshape(ne, F)
    qr = jnp.broadcast_to(q.reshape(nb, 1, F), (nb, K1, F)).reshape(ne, F)
    euv = euv_ref[...]
    x1 = ea1_ref[...]
    h = jnp.dot(x1, w1a_ref[...], preferred_element_type=jnp.float32)
    h += euv[:, 0:1] * pr + euv[:, 1:2] * qr
    h = _selu(h + b1_ref[...])
    h = _selu(jnp.dot(h, w2_ref[...], preferred_element_type=jnp.float32) + b2_ref[...])
    h = jnp.dot(h, w3_ref[...], preferred_element_type=jnp.float32) + b3_ref[...]
    mu = jnp.mean(h, axis=1, keepdims=True)
    d = h - mu
    var = jnp.mean(d * d, axis=1, keepdims=True)
    out_ref[...] = x1 + d * jax.lax.rsqrt(var + 1e-5) * g_ref[...] + bt_ref[...]


def _mlp_seg(v1_s, euv1, ea1, consts, grid, off, prev_out=None):
    nb = 200
    ne = nb * K1
    const = lambda i: (0, 0)
    in_specs = [
        pl.BlockSpec((nb, 2 * F), lambda i: (i, 0)),
        pl.BlockSpec((ne, 2), lambda i: (i + off, 0)),
        pl.BlockSpec((ne, F), lambda i: (i + off, 0)),
        pl.BlockSpec((F, F), const),
        pl.BlockSpec((F, F), const),
        pl.BlockSpec((1, F), const),
        pl.BlockSpec((F, F), const),
        pl.BlockSpec((1, F), const),
        pl.BlockSpec((F, F), const),
        pl.BlockSpec((1, F), const),
        pl.BlockSpec((1, F), const),
        pl.BlockSpec((1, F), const),
    ]
    args = [v1_s, euv1, ea1, *consts]
    kwargs = {}
    if prev_out is not None:
        in_specs = [pl.BlockSpec(memory_space=pl.ANY)] + in_specs
        args = [prev_out] + args
        kwargs["input_output_aliases"] = {0: 0}
    return pl.pallas_call(
        functools.partial(_mlp_body, nb=nb),
        grid=(grid,),
        in_specs=in_specs,
        out_specs=pl.BlockSpec((ne, F), lambda i: (i + off, 0)),
        out_shape=jax.ShapeDtypeStruct((E1, F), jnp.float32),
        **kwargs,
    )(*args)


def kernel(pos, y_idx_21, x_idx_21, weights_21, edge_attr2, edge_index2,
           edgeUnitVectorInverse2, coarse_mask2, edge_attr1, edge_index1,
           edgeUnitVector1, W1, b1, W2, b2, W3, b3, gamma, beta):
    ea2_3d = edge_attr2.reshape(V2, K2, F)
    euvt = edgeUnitVectorInverse2.transpose(0, 2, 1)
    v = _edge_to_node(euvt, ea2_3d)
    v2d = v.reshape(V2, 2 * F)

    idx = x_idx_21.astype(jnp.int32)
    w = weights_21.reshape(-1)
    v1s = []
    n0 = 0
    for seg in _SEGS:
        v1s.append(_knn_interp(v2d, idx[n0 * KI:(n0 + seg) * KI],
                               w[n0 * KI:(n0 + seg) * KI], seg))
        n0 += seg

    consts = (W1[:F], W1[F:], b1.reshape(1, F), W2, b2.reshape(1, F),
              W3, b3.reshape(1, F), gamma.reshape(1, F), beta.reshape(1, F))
    out = None
    off = 0
    for v1_s, seg in zip(v1s, _SEGS):
        grid = seg // 200
        out = _mlp_seg(v1_s, edgeUnitVector1, edge_attr1, consts, grid, off,
                       prev_out=out)
        off += grid
    return out
